# Initial kernel scaffold; baseline (speedup 1.0000x reference)
import jax
import jax.numpy as jnp
from jax.experimental import pallas as pl


def kernel(query_seq, matching_set, topk):
    raise NotImplementedError("write your pallas kernel here")


# speedup vs baseline: 2.3947x; 2.3947x over previous
import functools

import jax
import jax.numpy as jnp
from jax import lax
from jax.experimental import pallas as pl
from jax.experimental.pallas import tpu as pltpu
from jax.experimental.pallas import tpu_sc as plsc

Q = 1024
M = 16384
D = 1024
K = 4
BM = 1024
NM = M // BM

_NEG = float("-inf")
_BIG = 2**30


def _extract4(v, gi):
    vals, idxs = [], []
    for _ in range(K):
        m = jnp.max(v, axis=1, keepdims=True)
        cand = jnp.where(v == m, gi, _BIG)
        j = jnp.min(cand, axis=1, keepdims=True)
        vals.append(m)
        idxs.append(j)
        v = jnp.where(gi == j, _NEG, v)
    return jnp.concatenate(vals, axis=1), jnp.concatenate(idxs, axis=1)


def _topk_body(q_ref, mb_ref, qn_ref, mn_ref, out_ref, rv_ref, ri_ref):
    step = pl.program_id(0)

    @pl.when(step == 0)
    def _init():
        rv_ref[...] = jnp.full((Q, K), _NEG, jnp.float32)
        ri_ref[...] = jnp.full((Q, K), -1, jnp.int32)

    dot = lax.dot_general(
        q_ref[...], mb_ref[...],
        (((1,), (1,)), ((), ())),
        preferred_element_type=jnp.float32,
    )
    qn = qn_ref[...]
    mn = mn_ref[...]
    qn2 = qn * qn
    mn2 = mn * mn
    sq = (qn2 + mn2) - dot * 2.0
    c = jnp.maximum(0.0, sq)
    dotprod = ((-c) + qn2 + mn2) * 0.5
    r = pl.reciprocal(qn * mn, approx=True, full_range=False)
    dists = 1.0 - dotprod * r
    neg = -dists

    col = lax.broadcasted_iota(jnp.int32, (Q, BM), 1) + step * BM
    bv, bi = _extract4(neg, col)

    mv = jnp.concatenate([rv_ref[...], bv], axis=1)
    mi = jnp.concatenate([ri_ref[...], bi], axis=1)
    nv, ni = _extract4(mv, mi)
    rv_ref[...] = nv
    ri_ref[...] = ni

    @pl.when(step == NM - 1)
    def _done():
        out_ref[...] = ni


def _topk_indices(query_seq, matching_set, qn, mn):
    return pl.pallas_call(
        _topk_body,
        grid=(NM,),
        in_specs=[
            pl.BlockSpec((Q, D), lambda i: (0, 0)),
            pl.BlockSpec((BM, D), lambda i: (i, 0)),
            pl.BlockSpec((Q, 1), lambda i: (0, 0)),
            pl.BlockSpec((1, BM), lambda i: (0, i)),
        ],
        out_specs=pl.BlockSpec((Q, K), lambda i: (0, 0)),
        out_shape=jax.ShapeDtypeStruct((Q, K), jnp.int32),
        scratch_shapes=[
            pltpu.VMEM((Q, K), jnp.float32),
            pltpu.VMEM((Q, K), jnp.int32),
        ],
        compiler_params=pltpu.CompilerParams(
            dimension_semantics=("arbitrary",),
        ),
    )(query_seq, matching_set, qn, mn)


_NC = 2
_NS = 16
_NW = _NC * _NS
_QPW = Q // _NW
_QPI = 8
_NIT = _QPW // _QPI


def _gather_mean(matching_set, idx_flat):
    mesh = plsc.VectorSubcoreMesh(core_axis_name="c", subcore_axis_name="s")

    @functools.partial(
        pl.kernel,
        mesh=mesh,
        out_type=jax.ShapeDtypeStruct((Q, D), jnp.float32),
        scratch_types=[
            pltpu.VMEM((_QPI * K,), jnp.int32),
            pltpu.VMEM((_QPI * K, D), jnp.float32),
            pltpu.VMEM((_QPI, D), jnp.float32),
            pltpu.SemaphoreType.DMA,
        ],
    )
    def k(mset_hbm, idx_hbm, out_hbm, idx_v, rows_v, acc_v, sem):
        wid = lax.axis_index("s") * _NC + lax.axis_index("c")

        def body(i, carry):
            q0 = wid * _QPW + i * _QPI
            pltpu.sync_copy(idx_hbm.at[pl.ds(q0 * K, _QPI * K)], idx_v)
            pltpu.async_copy(mset_hbm.at[idx_v], rows_v, sem).wait()

            def chunk(c, carry2):
                for r in range(_QPI):
                    sl = pl.ds(c * 16, 16)
                    a = rows_v[K * r + 0, sl] + rows_v[K * r + 1, sl]
                    a = a + rows_v[K * r + 2, sl]
                    a = a + rows_v[K * r + 3, sl]
                    acc_v[r, sl] = a / 4.0
                return carry2

            lax.fori_loop(0, D // 16, chunk, 0)
            pltpu.sync_copy(acc_v, out_hbm.at[pl.ds(q0, _QPI)])
            return carry

        lax.fori_loop(0, _NIT, body, 0)

    return k(matching_set, idx_flat)


def kernel(query_seq, matching_set, topk):
    del topk
    qn = jnp.linalg.norm(query_seq, axis=-1)
    mn = jnp.linalg.norm(matching_set, axis=-1)
    idx = _topk_indices(
        query_seq, matching_set,
        qn.reshape(Q, 1), mn.reshape(1, M),
    )
    out = _gather_mean(matching_set, idx.reshape(Q * K))
    return out


# speedup vs baseline: 3.2039x; 1.3379x over previous
import functools

import jax
import jax.numpy as jnp
from jax import lax
from jax.experimental import pallas as pl
from jax.experimental.pallas import tpu as pltpu
from jax.experimental.pallas import tpu_sc as plsc

Q = 1024
M = 16384
D = 1024
K = 4
BM = 2048
NM = M // BM

_NEG = float("-inf")
_BIG = float("inf")


def _extract4(v, gi):
    vals, idxs = [], []
    for _ in range(K):
        m = jnp.max(v, axis=1, keepdims=True)
        cand = jnp.where(v == m, gi, _BIG)
        j = jnp.min(cand, axis=1, keepdims=True)
        vals.append(m)
        idxs.append(j)
        v = jnp.where(gi == j, _NEG, v)
    return jnp.concatenate(vals, axis=1), jnp.concatenate(idxs, axis=1)


def _topk_body(q_ref, mb_ref, qn_ref, mn_ref, out_ref, rv_ref, ri_ref):
    step = pl.program_id(0)

    @pl.when(step == 0)
    def _init():
        rv_ref[...] = jnp.full((Q, K), _NEG, jnp.float32)
        ri_ref[...] = jnp.full((Q, K), -1.0, jnp.float32)

    dot = lax.dot_general(
        q_ref[...], mb_ref[...],
        (((1,), (1,)), ((), ())),
        preferred_element_type=jnp.float32,
    )
    qn = qn_ref[...]
    mn = mn_ref[...]
    qn2 = qn * qn
    mn2 = mn * mn
    sq = (qn2 + mn2) - dot * 2.0
    c = jnp.maximum(0.0, sq)
    dotprod = ((-c) + qn2 + mn2) * 0.5
    r = pl.reciprocal(qn * mn, approx=True, full_range=False)
    neg = dotprod * r - 1.0

    col = (lax.broadcasted_iota(jnp.int32, (Q, BM), 1).astype(jnp.float32)
           + (step * BM).astype(jnp.float32))
    bv, bi = _extract4(neg, col)

    mv = jnp.concatenate([rv_ref[...], bv], axis=1)
    mi = jnp.concatenate([ri_ref[...], bi], axis=1)
    nv, ni = _extract4(mv, mi)
    rv_ref[...] = nv
    ri_ref[...] = ni

    @pl.when(step == NM - 1)
    def _done():
        out_ref[...] = ni.astype(jnp.int32)


def _topk_indices(query_seq, matching_set, qn, mn):
    return pl.pallas_call(
        _topk_body,
        grid=(NM,),
        in_specs=[
            pl.BlockSpec((Q, D), lambda i: (0, 0)),
            pl.BlockSpec((BM, D), lambda i: (i, 0)),
            pl.BlockSpec((Q, 1), lambda i: (0, 0)),
            pl.BlockSpec((1, BM), lambda i: (0, i)),
        ],
        out_specs=pl.BlockSpec((Q, K), lambda i: (0, 0)),
        out_shape=jax.ShapeDtypeStruct((Q, K), jnp.int32),
        scratch_shapes=[
            pltpu.VMEM((Q, K), jnp.float32),
            pltpu.VMEM((Q, K), jnp.float32),
        ],
        compiler_params=pltpu.CompilerParams(
            dimension_semantics=("arbitrary",),
        ),
    )(query_seq, matching_set, qn, mn)


_NC = 2
_NS = 16
_NW = _NC * _NS
_QPW = Q // _NW
_QPI = 8
_NIT = _QPW // _QPI


def _gather_mean(matching_set, idx_flat):
    mesh = plsc.VectorSubcoreMesh(core_axis_name="c", subcore_axis_name="s")

    @functools.partial(
        pl.kernel,
        mesh=mesh,
        out_type=jax.ShapeDtypeStruct((Q, D), jnp.float32),
        scratch_types=[
            pltpu.VMEM((_QPI * K,), jnp.int32),
            pltpu.VMEM((_QPI * K, D), jnp.float32),
            pltpu.VMEM((_QPI, D), jnp.float32),
            pltpu.SemaphoreType.DMA,
        ],
    )
    def k(mset_hbm, idx_hbm, out_hbm, idx_v, rows_v, acc_v, sem):
        wid = lax.axis_index("s") * _NC + lax.axis_index("c")

        def body(i, carry):
            q0 = wid * _QPW + i * _QPI
            pltpu.sync_copy(idx_hbm.at[pl.ds(q0 * K, _QPI * K)], idx_v)
            pltpu.async_copy(mset_hbm.at[idx_v], rows_v, sem).wait()

            def chunk(c, carry2):
                for r in range(_QPI):
                    sl = pl.ds(c * 16, 16)
                    a = rows_v[K * r + 0, sl] + rows_v[K * r + 1, sl]
                    a = a + rows_v[K * r + 2, sl]
                    a = a + rows_v[K * r + 3, sl]
                    acc_v[r, sl] = a / 4.0
                return carry2

            lax.fori_loop(0, D // 16, chunk, 0)
            pltpu.sync_copy(acc_v, out_hbm.at[pl.ds(q0, _QPI)])
            return carry

        lax.fori_loop(0, _NIT, body, 0)

    return k(matching_set, idx_flat)


def kernel(query_seq, matching_set, topk):
    del topk
    qn = jnp.linalg.norm(query_seq, axis=-1)
    mn = jnp.linalg.norm(matching_set, axis=-1)
    idx = _topk_indices(
        query_seq, matching_set,
        qn.reshape(Q, 1), mn.reshape(1, M),
    )
    out = _gather_mean(matching_set, idx.reshape(Q * K))
    return out
